# baseline (device time: 12619 ns/iter reference)
import jax
import jax.numpy as jnp
from jax import lax
from jax.experimental import pallas as pl
from jax.experimental.pallas import tpu as pltpu

N_DEV = 8


def kernel(x, w_mat):
    m_total, k_shard = x.shape
    k_total, n = w_mat.shape
    m_per = m_total // N_DEV

    def body(x_ref, w_ref, out_ref, x_bf, recv_buf, send_sems, recv_sems):
        my = lax.axis_index("i")

        x_bf[:, :] = x_ref[:, :].astype(jnp.bfloat16)

        barrier = pltpu.get_barrier_semaphore()
        for d in range(1, N_DEV):
            pl.semaphore_signal(
                barrier, inc=1,
                device_id=((my + d) % N_DEV,),
                device_id_type=pl.DeviceIdType.MESH,
            )
        pl.semaphore_wait(barrier, N_DEV - 1)

        rdmas = []
        for d in range(1, N_DEV):
            tgt = (my + d) % N_DEV
            rdma = pltpu.make_async_remote_copy(
                src_ref=x_bf.at[pl.ds(tgt * m_per, m_per), :],
                dst_ref=recv_buf.at[d - 1],
                send_sem=send_sems.at[d - 1],
                recv_sem=recv_sems.at[d - 1],
                device_id=(tgt,),
                device_id_type=pl.DeviceIdType.MESH,
            )
            rdma.start()
            rdmas.append(rdma)

        own = x_bf[pl.ds(my * m_per, m_per), :]
        w_own = w_ref[pl.ds(my * k_shard, k_shard), :].astype(jnp.bfloat16)
        acc = jnp.dot(own, w_own, preferred_element_type=jnp.float32)

        for d in range(1, N_DEV):
            src = (my + N_DEV - d) % N_DEV
            rdmas[d - 1].wait_recv()
            w_blk = w_ref[pl.ds(src * k_shard, k_shard), :].astype(jnp.bfloat16)
            acc = acc + jnp.dot(
                recv_buf[d - 1], w_blk, preferred_element_type=jnp.float32
            )

        for d in range(1, N_DEV):
            rdmas[d - 1].wait_send()

        out_ref[:, :] = acc * jax.nn.sigmoid(acc)

    return pl.pallas_call(
        body,
        out_shape=jax.ShapeDtypeStruct((m_per, n), jnp.float32),
        in_specs=[
            pl.BlockSpec(memory_space=pltpu.VMEM),
            pl.BlockSpec(memory_space=pltpu.VMEM),
        ],
        out_specs=pl.BlockSpec(memory_space=pltpu.VMEM),
        scratch_shapes=[
            pltpu.VMEM((m_total, k_shard), jnp.bfloat16),
            pltpu.VMEM((N_DEV - 1, m_per, k_shard), jnp.bfloat16),
            pltpu.SemaphoreType.DMA((N_DEV - 1,)),
            pltpu.SemaphoreType.DMA((N_DEV - 1,)),
        ],
        compiler_params=pltpu.CompilerParams(collective_id=0),
    )(x, w_mat)


# device time: 12478 ns/iter; 1.0113x vs baseline; 1.0113x over previous
import jax
import jax.numpy as jnp
from jax import lax
from jax.experimental import pallas as pl
from jax.experimental.pallas import tpu as pltpu

N_DEV = 8


def kernel(x, w_mat):
    m_total, k_shard = x.shape
    k_total, n = w_mat.shape
    m_per = m_total // N_DEV

    def body(x_ref, w_ref, out_ref, x_bf, x_full, w_bf, send_sems, recv_sems):
        my = lax.axis_index("i")

        x_bf[:, :] = x_ref[:, :].astype(jnp.bfloat16)

        barrier = pltpu.get_barrier_semaphore()
        for d in range(1, N_DEV):
            pl.semaphore_signal(
                barrier, inc=1,
                device_id=((my + d) % N_DEV,),
                device_id_type=pl.DeviceIdType.MESH,
            )
        pl.semaphore_wait(barrier, N_DEV - 1)

        sends = []
        for d in range(1, N_DEV):
            tgt = (my + d) % N_DEV
            rdma = pltpu.make_async_remote_copy(
                src_ref=x_bf.at[pl.ds(tgt * m_per, m_per), :],
                dst_ref=x_full.at[:, pl.ds(my * k_shard, k_shard)],
                send_sem=send_sems.at[d - 1],
                recv_sem=recv_sems.at[d - 1],
                device_id=(tgt,),
                device_id_type=pl.DeviceIdType.MESH,
            )
            rdma.start()
            sends.append(rdma)

        x_full[:, pl.ds(my * k_shard, k_shard)] = x_bf[pl.ds(my * m_per, m_per), :]
        w_bf[:, :] = w_ref[:, :].astype(jnp.bfloat16)

        for d in range(1, N_DEV):
            src = (my + N_DEV - d) % N_DEV
            recv = pltpu.make_async_remote_copy(
                src_ref=x_bf.at[pl.ds(my * m_per, m_per), :],
                dst_ref=x_full.at[:, pl.ds(src * k_shard, k_shard)],
                send_sem=send_sems.at[d - 1],
                recv_sem=recv_sems.at[d - 1],
                device_id=(src,),
                device_id_type=pl.DeviceIdType.MESH,
            )
            recv.wait_recv()

        acc = jnp.dot(x_full[:, :], w_bf[:, :], preferred_element_type=jnp.float32)
        out_ref[:, :] = acc * jax.nn.sigmoid(acc)

        for s in sends:
            s.wait_send()

    return pl.pallas_call(
        body,
        out_shape=jax.ShapeDtypeStruct((m_per, n), jnp.float32),
        in_specs=[
            pl.BlockSpec(memory_space=pltpu.VMEM),
            pl.BlockSpec(memory_space=pltpu.VMEM),
        ],
        out_specs=pl.BlockSpec(memory_space=pltpu.VMEM),
        scratch_shapes=[
            pltpu.VMEM((m_total, k_shard), jnp.bfloat16),
            pltpu.VMEM((m_per, k_total), jnp.bfloat16),
            pltpu.VMEM((k_total, n), jnp.bfloat16),
            pltpu.SemaphoreType.DMA((N_DEV - 1,)),
            pltpu.SemaphoreType.DMA((N_DEV - 1,)),
        ],
        compiler_params=pltpu.CompilerParams(collective_id=0),
    )(x, w_mat)


# device time: 12418 ns/iter; 1.0162x vs baseline; 1.0048x over previous
import jax
import jax.numpy as jnp
from jax import lax
from jax.experimental import pallas as pl
from jax.experimental.pallas import tpu as pltpu

N_DEV = 8


def kernel(x, w_mat):
    m_total, k_shard = x.shape
    k_total, n = w_mat.shape
    m_per = m_total // N_DEV

    def body(x_ref, w_ref, out_ref, x_bf, x_full, w_bf,
             send_sems, recv_sems, local_sem):
        my = lax.axis_index("i")

        barrier = pltpu.get_barrier_semaphore()
        for d in range(1, N_DEV):
            pl.semaphore_signal(
                barrier, inc=1,
                device_id=((my + d) % N_DEV,),
                device_id_type=pl.DeviceIdType.MESH,
            )

        x_bf[:, :] = x_ref[:, :].astype(jnp.bfloat16)
        own = pltpu.make_async_copy(
            x_bf.at[pl.ds(my * m_per, m_per), :],
            x_full.at[:, pl.ds(my * k_shard, k_shard)],
            local_sem,
        )
        own.start()
        w_bf[:, :] = w_ref[:, :].astype(jnp.bfloat16)

        pl.semaphore_wait(barrier, N_DEV - 1)

        sends = []
        for d in range(1, N_DEV):
            tgt = (my + d) % N_DEV
            rdma = pltpu.make_async_remote_copy(
                src_ref=x_bf.at[pl.ds(tgt * m_per, m_per), :],
                dst_ref=x_full.at[:, pl.ds(my * k_shard, k_shard)],
                send_sem=send_sems.at[d - 1],
                recv_sem=recv_sems.at[d - 1],
                device_id=(tgt,),
                device_id_type=pl.DeviceIdType.MESH,
            )
            rdma.start()
            sends.append(rdma)

        own.wait()
        for d in range(1, N_DEV):
            src = (my + N_DEV - d) % N_DEV
            recv = pltpu.make_async_remote_copy(
                src_ref=x_bf.at[pl.ds(my * m_per, m_per), :],
                dst_ref=x_full.at[:, pl.ds(src * k_shard, k_shard)],
                send_sem=send_sems.at[d - 1],
                recv_sem=recv_sems.at[d - 1],
                device_id=(src,),
                device_id_type=pl.DeviceIdType.MESH,
            )
            recv.wait_recv()

        acc = jnp.dot(x_full[:, :], w_bf[:, :], preferred_element_type=jnp.float32)
        out_ref[:, :] = acc * jax.nn.sigmoid(acc)

        for s in sends:
            s.wait_send()

    return pl.pallas_call(
        body,
        out_shape=jax.ShapeDtypeStruct((m_per, n), jnp.float32),
        in_specs=[
            pl.BlockSpec(memory_space=pltpu.VMEM),
            pl.BlockSpec(memory_space=pltpu.VMEM),
        ],
        out_specs=pl.BlockSpec(memory_space=pltpu.VMEM),
        scratch_shapes=[
            pltpu.VMEM((m_total, k_shard), jnp.bfloat16),
            pltpu.VMEM((m_per, k_total), jnp.bfloat16),
            pltpu.VMEM((k_total, n), jnp.bfloat16),
            pltpu.SemaphoreType.DMA((N_DEV - 1,)),
            pltpu.SemaphoreType.DMA((N_DEV - 1,)),
            pltpu.SemaphoreType.DMA,
        ],
        compiler_params=pltpu.CompilerParams(collective_id=0),
    )(x, w_mat)
